# baseline (speedup 1.0000x reference)
import functools

import jax
import jax.numpy as jnp
from jax import lax
from jax.experimental import pallas as pl
from jax.experimental.pallas import tpu as pltpu
from jax.experimental.pallas import tpu_sc as plsc

F32 = jnp.float32
NC, NS = 2, 16
NW = NC * NS
CHUNK = 80
BR = 1024


def _ceil_to(x, m):
    return (x + m - 1) // m * m


def _pack_bf16(x):
    k = x.shape[1] // 2
    xi = jax.lax.bitcast_convert_type(x, jnp.int32)
    lo = jax.lax.shift_right_logical(xi[:, :k] + 0x8000, 16)
    hi = (xi[:, k:] + 0x8000) & jnp.int32(-65536)
    return lo | hi


def _unpack_bf16(gi):
    a = jax.lax.bitcast_convert_type(gi << 16, F32)
    b = jax.lax.bitcast_convert_type(gi & jnp.int32(-65536), F32)
    return jnp.concatenate([a, b], axis=1)


def _mlp3_body(x, w1, b1, w2, b2, w3, b3, out):
    h = jnp.maximum(jnp.dot(x[...], w1[0], preferred_element_type=F32) + b1[0], 0.0)
    h = jnp.maximum(jnp.dot(h, w2[0], preferred_element_type=F32) + b2[0], 0.0)
    out[...] = jnp.dot(h, w3[0], preferred_element_type=F32) + b3[0]


def _mlp3_call(x, ws, n_first):
    n, din = x.shape
    dout = ws[4].shape[-1]
    grid = (n // BR,)
    nb_first = n_first // BR

    def wsel(i):
        return (jnp.where(i < nb_first, 0, 1), 0, 0)

    wspecs = []
    for a in ws:
        s, d0, d1 = a.shape
        wspecs.append(pl.BlockSpec((1, d0, d1), wsel if s > 1 else (lambda i: (0, 0, 0))))
    return pl.pallas_call(
        _mlp3_body,
        grid=grid,
        in_specs=[pl.BlockSpec((BR, din), lambda i: (i, 0))] + wspecs,
        out_specs=pl.BlockSpec((BR, dout), lambda i: (i, 0)),
        out_shape=jax.ShapeDtypeStruct((n, dout), F32),
    )(x, *ws)


def _proj_body(h, wa, wb, ps, pd):
    ps[...] = jnp.dot(h[...], wa[...], preferred_element_type=F32)
    pd[...] = jnp.dot(h[...], wb[...], preferred_element_type=F32)


def _proj_call(h, wa, wb):
    n, d = h.shape
    return pl.pallas_call(
        _proj_body,
        grid=(n // BR,),
        in_specs=[pl.BlockSpec((BR, d), lambda i: (i, 0)),
                  pl.BlockSpec((d, d), lambda i: (0, 0)),
                  pl.BlockSpec((d, d), lambda i: (0, 0))],
        out_specs=[pl.BlockSpec((BR, d), lambda i: (i, 0)),
                   pl.BlockSpec((BR, d), lambda i: (i, 0))],
        out_shape=[jax.ShapeDtypeStruct((n, d), F32),
                   jax.ShapeDtypeStruct((n, d), F32)],
    )(h, wa, wb)


def _edge_body(g1, g2, e, w1c, b1, w2, b2, w3, b3, out):
    a = jnp.maximum(_unpack_bf16(g1[...]) + _unpack_bf16(g2[...]) + b1[...]
                    + jnp.dot(e[...], w1c[...], preferred_element_type=F32), 0.0)
    a = jnp.maximum(jnp.dot(a, w2[...], preferred_element_type=F32) + b2[...], 0.0)
    out[...] = e[...] + jnp.dot(a, w3[...], preferred_element_type=F32) + b3[...]


def _edge_call(g1, g2, e, w1c, b1, w2, b2, w3, b3):
    n, d = e.shape
    row = pl.BlockSpec((BR, d), lambda i: (i, 0))
    prow = pl.BlockSpec((BR, d // 2), lambda i: (i, 0))
    mat = pl.BlockSpec((d, d), lambda i: (0, 0))
    vec = pl.BlockSpec((1, d), lambda i: (0, 0))
    return pl.pallas_call(
        _edge_body,
        grid=(n // BR,),
        in_specs=[prow, prow, row, mat, vec, mat, vec, mat, vec],
        out_specs=row,
        out_shape=jax.ShapeDtypeStruct((n, d), F32),
    )(g1, g2, e, w1c, b1, w2, b2, w3, b3)


def _node_body(h, am0, am1, ac0, ac1, im, ic, w1a, w1b, w1c, b1, w2, b2, w3, b3,
               wea, web, hn_out, ps_out, pd_out):
    am_n = (am0[0] + am1[0]) * im[...]
    ac_n = (ac0[0] + ac1[0]) * ic[...]
    a = (jnp.dot(h[...], w1a[...], preferred_element_type=F32)
         + jnp.dot(am_n, w1b[...], preferred_element_type=F32)
         + jnp.dot(ac_n, w1c[...], preferred_element_type=F32) + b1[...])
    a = jnp.maximum(a, 0.0)
    a = jnp.maximum(jnp.dot(a, w2[...], preferred_element_type=F32) + b2[...], 0.0)
    hn = h[...] + jnp.dot(a, w3[...], preferred_element_type=F32) + b3[...]
    hn_out[...] = hn
    ps_out[...] = jnp.dot(hn, wea[...], preferred_element_type=F32)
    pd_out[...] = jnp.dot(hn, web[...], preferred_element_type=F32)


def _node_call(h, am, ac, im, ic, w1a, w1b, w1c, b1, w2, b2, w3, b3, wea, web):
    n, d = h.shape
    row = pl.BlockSpec((BR, d), lambda i: (i, 0))
    prow0 = pl.BlockSpec((1, BR, d), lambda i: (0, i, 0))
    prow1 = pl.BlockSpec((1, BR, d), lambda i: (1, i, 0))
    mat = pl.BlockSpec((d, d), lambda i: (0, 0))
    vec = pl.BlockSpec((1, d), lambda i: (0, 0))
    return pl.pallas_call(
        _node_body,
        grid=(n // BR,),
        in_specs=[row, prow0, prow1, prow0, prow1, row, row,
                  mat, mat, mat, vec, mat, vec, mat, vec, mat, mat],
        out_specs=[row, row, row],
        out_shape=[jax.ShapeDtypeStruct((n, d), F32)] * 3,
    )(h, am, am, ac, ac, im, ic, w1a, w1b, w1c, b1, w2, b2, w3, b3, wea, web)


def _sc_mesh():
    return plsc.VectorSubcoreMesh(core_axis_name="c", subcore_axis_name="s",
                                  num_cores=NC, num_subcores=NS)


def _pack_rows(rv, pv):
    def row(r, _):
        for t in range(4):
            x = rv[r, pl.ds(16 * t, 16)]
            y = rv[r, pl.ds(64 + 16 * t, 16)]
            pv[r, pl.ds(16 * t, 16)] = (
                jax.lax.shift_right_logical(x + 0x8000, 16)
                | ((y + 0x8000) & jnp.int32(-65536)))
        return 0
    lax.fori_loop(0, CHUNK, row, 0)


def _gather_pipe(np_rows, idx_hbm, tbl, out, hbm_dummy, ix_a, ix_b, rv_a, rv_b,
                 pv_a, pv_b, gs_a, gs_b, ws_a, ws_b, tid):
    e_total = out.shape[0]
    ept = e_total // NS
    nch = ept // CHUNK
    base0 = tid * ept
    dummy = out.at[pl.ds(0, CHUNK)]

    def idx_load(j, ix):
        pltpu.sync_copy(idx_hbm.at[pl.ds(base0 + j * CHUNK, CHUNK)], ix)

    def fire_gather(ix, rv, sem):
        pltpu.async_copy(tbl.at[ix], rv, sem)

    def wait1(sem, rv):
        pltpu.make_async_copy(hbm_dummy, rv, sem).wait()

    def fire_write(j, pv, sem):
        pltpu.async_copy(pv, out.at[pl.ds(base0 + j * CHUNK, CHUNK)], sem)

    def wait_w(sem, pv):
        pltpu.make_async_copy(dummy, pv, sem).wait()

    idx_load(0, ix_a)
    fire_gather(ix_a, rv_a, gs_a)
    idx_load(1, ix_b)

    def body(g, _):
        j = 2 * g
        wait1(gs_a, rv_a)

        @pl.when(g > 0)
        def _():
            wait_w(ws_b, pv_b)
        fire_gather(ix_b, rv_b, gs_b)
        _pack_rows(rv_a, pv_a)
        fire_write(j, pv_a, ws_a)

        @pl.when(j + 2 < nch)
        def _():
            idx_load(j + 2, ix_a)
        wait1(gs_b, rv_b)
        wait_w(ws_a, pv_a)

        @pl.when(j + 2 < nch)
        def _():
            fire_gather(ix_a, rv_a, gs_a)
        _pack_rows(rv_b, pv_b)
        fire_write(j + 1, pv_b, ws_b)

        @pl.when(j + 3 < nch)
        def _():
            idx_load(j + 3, ix_b)
        return 0

    lax.fori_loop(0, nch // 2, body, 0)
    wait_w(ws_b, pv_b)


def _gather_body(np_rows, tbl_rows, ps, pd, src, dst, g1, g2,
                 ix_a, ix_b, rv_a, rv_b, pv_a, pv_b, tbl, gs_a, gs_b, ws_a, ws_b):
    core = lax.axis_index("c")
    tid = lax.axis_index("s")
    trows = tbl_rows // NS

    @pl.when(core == 0)
    def _():
        pltpu.sync_copy(ps.at[pl.ds(tid * trows, trows)],
                        tbl.at[pl.ds(tid * trows, trows), :])

    @pl.when(core == 1)
    def _():
        pltpu.sync_copy(pd.at[pl.ds(tid * trows, trows)],
                        tbl.at[pl.ds(tid * trows, trows), :])
    plsc.subcore_barrier()

    hbm_dummy = ps.at[pl.ds(0, CHUNK)]

    @pl.when(core == 0)
    def _():
        _gather_pipe(np_rows, src, tbl, g1, hbm_dummy, ix_a, ix_b, rv_a, rv_b,
                     pv_a, pv_b, gs_a, gs_b, ws_a, ws_b, tid)

    @pl.when(core == 1)
    def _():
        _gather_pipe(np_rows, dst, tbl, g2, hbm_dummy, ix_a, ix_b, rv_a, rv_b,
                     pv_a, pv_b, gs_a, gs_b, ws_a, ws_b, tid)


def _gather_call(e_total, d, np_rows, tbl_rows):
    return pl.kernel(
        functools.partial(_gather_body, np_rows, tbl_rows),
        out_type=[jax.ShapeDtypeStruct((e_total, d // 2), jnp.int32),
                  jax.ShapeDtypeStruct((e_total, d // 2), jnp.int32)],
        mesh=_sc_mesh(),
        scratch_types=[
            pltpu.VMEM((CHUNK,), jnp.int32),
            pltpu.VMEM((CHUNK,), jnp.int32),
            pltpu.VMEM((CHUNK, d), jnp.int32),
            pltpu.VMEM((CHUNK, d), jnp.int32),
            pltpu.VMEM((CHUNK, d // 2), jnp.int32),
            pltpu.VMEM((CHUNK, d // 2), jnp.int32),
            pltpu.VMEM_SHARED((tbl_rows, d), jnp.int32),
            pltpu.SemaphoreType.DMA,
            pltpu.SemaphoreType.DMA,
            pltpu.SemaphoreType.DMA,
            pltpu.SemaphoreType.DMA,
        ],
    )


def _zero_block(buf):
    def zr(r, _):
        for c in range(8):
            buf[r, pl.ds(c * 16, 16)] = jnp.zeros((16,), F32)
        return 0
    lax.fori_loop(0, CHUNK, zr, 0)


def _ones_block(buf):
    def zr(r, _):
        for c in range(8):
            buf[r, pl.ds(c * 16, 16)] = jnp.ones((16,), F32)
        return 0
    lax.fori_loop(0, CHUNK, zr, 0)


def _scatter_body(base_off, n_edges, np_rows, acc_rows, e, dsts, out,
                  ix_a, ix_b, rv_a, rv_b, acc, ls_a, ls_b, ss_a, ss_b):
    core = lax.axis_index("c")
    tid = lax.axis_index("s")
    ept = n_edges // (NC * NS)
    nch = ept // CHUNK
    zrows = acc_rows // NS
    base0 = core * (n_edges // NC) + tid * ept
    dummy_i = dsts.at[pl.ds(0, CHUNK)]
    dummy_r = e.at[pl.ds(0, CHUNK), :]

    _zero_block(rv_a)
    nz, rem = zrows // CHUNK, zrows % CHUNK
    for k in range(nz):
        pltpu.sync_copy(rv_a, acc.at[pl.ds(tid * zrows + k * CHUNK, CHUNK), :])
    if rem:
        pltpu.sync_copy(rv_a.at[pl.ds(0, rem), :],
                        acc.at[pl.ds(tid * zrows + nz * CHUNK, rem), :])
    plsc.subcore_barrier()

    def load(j, ix, rv, sem):
        pltpu.async_copy(dsts.at[pl.ds(base0 + j * CHUNK, CHUNK)], ix, sem)
        pltpu.async_copy(e.at[pl.ds(base_off + base0 + j * CHUNK, CHUNK), :], rv, sem)

    def wait_load(sem, ix, rv):
        pltpu.make_async_copy(dummy_i, ix, sem).wait()
        pltpu.make_async_copy(dummy_r, rv, sem).wait()

    def wait_add(sem, rv):
        pltpu.make_async_copy(dummy_r, rv, sem).wait()

    load(0, ix_a, rv_a, ls_a)

    def body(g, _):
        j = 2 * g
        wait_load(ls_a, ix_a, rv_a)

        @pl.when(g > 0)
        def _():
            wait_add(ss_b, rv_b)
        load(j + 1, ix_b, rv_b, ls_b)
        pltpu.async_copy(rv_a, acc.at[ix_a], ss_a, add=True)
        wait_load(ls_b, ix_b, rv_b)
        wait_add(ss_a, rv_a)

        @pl.when(j + 2 < nch)
        def _():
            load(j + 2, ix_a, rv_a, ls_a)
        pltpu.async_copy(rv_b, acc.at[ix_b], ss_b, add=True)
        return 0

    lax.fori_loop(0, nch // 2, body, 0)
    wait_add(ss_b, rv_b)
    plsc.subcore_barrier()

    pltpu.sync_copy(acc.at[pl.ds(tid * zrows, zrows), :],
                    out.at[core, pl.ds(tid * zrows, zrows), :])


def _scatter_call(base_off, n_edges, d, np_rows, acc_rows):
    return pl.kernel(
        functools.partial(_scatter_body, base_off, n_edges, np_rows, acc_rows),
        out_type=jax.ShapeDtypeStruct((NC, np_rows, d), F32),
        mesh=_sc_mesh(),
        scratch_types=[
            pltpu.VMEM((CHUNK,), jnp.int32),
            pltpu.VMEM((CHUNK,), jnp.int32),
            pltpu.VMEM((CHUNK, 128), F32),
            pltpu.VMEM((CHUNK, 128), F32),
            pltpu.VMEM_SHARED((acc_rows, 128), F32),
            pltpu.SemaphoreType.DMA,
            pltpu.SemaphoreType.DMA,
            pltpu.SemaphoreType.DMA,
            pltpu.SemaphoreType.DMA,
        ],
    )


def _degree_body(n_edges, np_rows, acc_rows, dsts, out, idx_v, ones_v, zbuf, acc):
    core = lax.axis_index("c")
    tid = lax.axis_index("s")
    ept = n_edges // (NC * NS)
    nch = ept // CHUNK
    zrows = acc_rows // NS

    _zero_block(zbuf)
    nz, rem = zrows // CHUNK, zrows % CHUNK
    for k in range(nz):
        pltpu.sync_copy(zbuf, acc.at[pl.ds(tid * zrows + k * CHUNK, CHUNK), :])
    if rem:
        pltpu.sync_copy(zbuf.at[pl.ds(0, rem), :],
                        acc.at[pl.ds(tid * zrows + nz * CHUNK, rem), :])
    _ones_block(ones_v)
    plsc.subcore_barrier()

    def body(i, _):
        base = core * (n_edges // NC) + tid * ept + i * CHUNK
        pltpu.sync_copy(dsts.at[pl.ds(base, CHUNK)], idx_v)
        pltpu.sync_copy(ones_v, acc.at[idx_v], add=True)
        return 0

    lax.fori_loop(0, nch, body, 0)
    plsc.subcore_barrier()

    pltpu.sync_copy(acc.at[pl.ds(tid * zrows, zrows), :],
                    out.at[core, pl.ds(tid * zrows, zrows), :])


def _degree_call(n_edges, d, np_rows, acc_rows):
    return pl.kernel(
        functools.partial(_degree_body, n_edges, np_rows, acc_rows),
        out_type=jax.ShapeDtypeStruct((NC, np_rows, d), F32),
        mesh=_sc_mesh(),
        scratch_types=[
            pltpu.VMEM((CHUNK,), jnp.int32),
            pltpu.VMEM((CHUNK, 128), F32),
            pltpu.VMEM((CHUNK, 128), F32),
            pltpu.VMEM_SHARED((acc_rows, 128), F32),
        ],
    )


def _invdeg_body(dm0, dm1, dc0, dc1, om, oc):
    om[...] = 1.0 / jnp.clip(dm0[...] + dm1[...], 1.0, None)
    oc[...] = 1.0 / jnp.clip(dc0[...] + dc1[...], 1.0, None)


def _invdeg_call(dm, dc):
    n, d = dm.shape[1:]
    row = pl.BlockSpec((BR, d), lambda i: (i, 0))
    return pl.pallas_call(
        _invdeg_body,
        grid=(n // BR,),
        in_specs=[row, row, row, row],
        out_specs=[row, row],
        out_shape=[jax.ShapeDtypeStruct((n, d), F32)] * 2,
    )(dm[0], dm[1], dc[0], dc[1])


def kernel(x, mesh_edge_index, mesh_edge_attr, contact_edge_index, contact_edge_attr, params):
    n_nodes, node_dim = x.shape
    n_mesh = mesh_edge_attr.shape[0]
    n_contact = contact_edge_attr.shape[0]
    edge_dim = mesh_edge_attr.shape[1]
    d = params["node_enc"][4].shape[1]
    steps = 5

    np_rows = _ceil_to(n_nodes + 1, BR)
    acc_rows = _ceil_to(n_nodes + 1, 128)
    em = _ceil_to(n_mesh, 2 * NW * CHUNK)
    ec = _ceil_to(n_contact, 2 * NW * CHUNK)
    e_total = em + ec

    xp = jnp.pad(x, ((0, np_rows - n_nodes), (0, 0)))
    msrc = jnp.pad(mesh_edge_index[0], (0, em - n_mesh))
    mdst = jnp.pad(mesh_edge_index[1], (0, em - n_mesh), constant_values=n_nodes)
    csrc = jnp.pad(contact_edge_index[0], (0, ec - n_contact))
    cdst = jnp.pad(contact_edge_index[1], (0, ec - n_contact), constant_values=n_nodes)
    src_all = jnp.concatenate([msrc, csrc])
    dst_all = jnp.concatenate([mdst, cdst])
    ea = jnp.concatenate([
        jnp.pad(mesh_edge_attr, ((0, em - n_mesh), (0, 0))),
        jnp.pad(contact_edge_attr, ((0, ec - n_contact), (0, 0))),
    ])

    def stack2(pa, pb):
        out = []
        for wa, wb in zip(pa, pb):
            if wa.ndim == 1:
                wa, wb = wa[None, :], wb[None, :]
            out.append(jnp.stack([wa, wb]))
        return out

    def stack1(p):
        return [a[None][:, None, :] if a.ndim == 1 else a[None] for a in p]

    enc_e_ws = stack2(params["mesh_enc"], params["contact_enc"])
    enc_n_ws = stack1(params["node_enc"])

    we1, be1, we2, be2, we3, be3 = params["edge_mlp"]
    we1a, we1b, we1c = we1[:d], we1[d:2 * d], we1[2 * d:]
    be1r, be2r, be3r = be1[None, :], be2[None, :], be3[None, :]

    wn1, bn1, wn2, bn2, wn3, bn3 = params["node_mlp"]
    wn1a, wn1b, wn1c = wn1[:d], wn1[d:2 * d], wn1[2 * d:]
    bn1r, bn2r, bn3r = bn1[None, :], bn2[None, :], bn3[None, :]

    wd1, bd1, wd2, bd2, wd3, bd3 = params["decoder"]
    out_dim = wd3.shape[1]
    wd3p = jnp.pad(wd3, ((0, 0), (0, d - out_dim)))
    bd3p = jnp.pad(bd3, (0, d - out_dim))
    dec_ws = stack1((wd1, bd1, wd2, bd2, wd3p, bd3p))

    gather = _gather_call(e_total, d, np_rows, acc_rows)
    scatter_m = _scatter_call(0, em, d, np_rows, acc_rows)
    scatter_c = _scatter_call(em, ec, d, np_rows, acc_rows)

    e = _mlp3_call(ea, enc_e_ws, em)
    h = _mlp3_call(xp, enc_n_ws, np_rows)
    ps, pd = _proj_call(h, we1a, we1b)
    degm = _degree_call(em, d, np_rows, acc_rows)(mdst)
    degc = _degree_call(ec, d, np_rows, acc_rows)(cdst)
    invm, invc = _invdeg_call(degm, degc)

    for _ in range(steps):
        ps_i = jax.lax.bitcast_convert_type(ps, jnp.int32)
        pd_i = jax.lax.bitcast_convert_type(pd, jnp.int32)
        g1, g2 = gather(ps_i, pd_i, src_all, dst_all)
        e = _edge_call(g1, g2, e, we1c, be1r, we2, be2r, we3, be3r)
        aggm = scatter_m(e, mdst)
        aggc = scatter_c(e, cdst)
        h, ps, pd = _node_call(h, aggm, aggc, invm, invc,
                               wn1a, wn1b, wn1c, bn1r, wn2, bn2r, wn3, bn3r,
                               we1a, we1b)

    dec = _mlp3_call(h, dec_ws, np_rows)
    return dec[:n_nodes, :out_dim]


# speedup vs baseline: 1.1409x; 1.1409x over previous
import functools

import jax
import jax.numpy as jnp
from jax import lax
from jax.experimental import pallas as pl
from jax.experimental.pallas import tpu as pltpu
from jax.experimental.pallas import tpu_sc as plsc

F32 = jnp.float32
NC, NS = 2, 16
NW = NC * NS
CHUNK = 128
BR = 1024


def _ceil_to(x, m):
    return (x + m - 1) // m * m


def _pack_bf16(x):
    k = x.shape[1] // 2
    xi = jax.lax.bitcast_convert_type(x, jnp.int32)
    lo = jax.lax.shift_right_logical(xi[:, :k] + 0x8000, 16)
    hi = (xi[:, k:] + 0x8000) & jnp.int32(-65536)
    return lo | hi


def _unpack_bf16(gi):
    a = jax.lax.bitcast_convert_type(gi << 16, F32)
    b = jax.lax.bitcast_convert_type(gi & jnp.int32(-65536), F32)
    return jnp.concatenate([a, b], axis=1)


def _mlp3_body(x, w1, b1, w2, b2, w3, b3, out):
    h = jnp.maximum(jnp.dot(x[...], w1[0], preferred_element_type=F32) + b1[0], 0.0)
    h = jnp.maximum(jnp.dot(h, w2[0], preferred_element_type=F32) + b2[0], 0.0)
    out[...] = jnp.dot(h, w3[0], preferred_element_type=F32) + b3[0]


def _mlp3_call(x, ws, n_first):
    n, din = x.shape
    dout = ws[4].shape[-1]
    grid = (n // BR,)
    nb_first = n_first // BR

    def wsel(i):
        return (jnp.where(i < nb_first, 0, 1), 0, 0)

    wspecs = []
    for a in ws:
        s, d0, d1 = a.shape
        wspecs.append(pl.BlockSpec((1, d0, d1), wsel if s > 1 else (lambda i: (0, 0, 0))))
    return pl.pallas_call(
        _mlp3_body,
        grid=grid,
        in_specs=[pl.BlockSpec((BR, din), lambda i: (i, 0))] + wspecs,
        out_specs=pl.BlockSpec((BR, dout), lambda i: (i, 0)),
        out_shape=jax.ShapeDtypeStruct((n, dout), F32),
    )(x, *ws)


def _proj_body(h, wa, wb, ps, pd):
    ps[...] = jnp.dot(h[...], wa[...], preferred_element_type=F32)
    pd[...] = jnp.dot(h[...], wb[...], preferred_element_type=F32)


def _proj_call(h, wa, wb):
    n, d = h.shape
    return pl.pallas_call(
        _proj_body,
        grid=(n // BR,),
        in_specs=[pl.BlockSpec((BR, d), lambda i: (i, 0)),
                  pl.BlockSpec((d, d), lambda i: (0, 0)),
                  pl.BlockSpec((d, d), lambda i: (0, 0))],
        out_specs=[pl.BlockSpec((BR, d), lambda i: (i, 0)),
                   pl.BlockSpec((BR, d), lambda i: (i, 0))],
        out_shape=[jax.ShapeDtypeStruct((n, d), F32),
                   jax.ShapeDtypeStruct((n, d), F32)],
    )(h, wa, wb)


def _edge_body(g1, g2, e, w1c, b1, w2, b2, w3, b3, out):
    a = jnp.maximum(g1[...] + g2[...] + b1[...]
                    + jnp.dot(e[...], w1c[...], preferred_element_type=F32), 0.0)
    a = jnp.maximum(jnp.dot(a, w2[...], preferred_element_type=F32) + b2[...], 0.0)
    out[...] = e[...] + jnp.dot(a, w3[...], preferred_element_type=F32) + b3[...]


def _edge_call(g1, g2, e, w1c, b1, w2, b2, w3, b3):
    n, d = e.shape
    row = pl.BlockSpec((BR, d), lambda i: (i, 0))
    mat = pl.BlockSpec((d, d), lambda i: (0, 0))
    vec = pl.BlockSpec((1, d), lambda i: (0, 0))
    return pl.pallas_call(
        _edge_body,
        grid=(n // BR,),
        in_specs=[row, row, row, mat, vec, mat, vec, mat, vec],
        out_specs=row,
        out_shape=jax.ShapeDtypeStruct((n, d), F32),
    )(g1, g2, e, w1c, b1, w2, b2, w3, b3)


def _node_body(h, am0, am1, ac0, ac1, im, ic, w1a, w1b, w1c, b1, w2, b2, w3, b3,
               wea, web, hn_out, ps_out, pd_out):
    am_n = (am0[0] + am1[0]) * im[...]
    ac_n = (ac0[0] + ac1[0]) * ic[...]
    a = (jnp.dot(h[...], w1a[...], preferred_element_type=F32)
         + jnp.dot(am_n, w1b[...], preferred_element_type=F32)
         + jnp.dot(ac_n, w1c[...], preferred_element_type=F32) + b1[...])
    a = jnp.maximum(a, 0.0)
    a = jnp.maximum(jnp.dot(a, w2[...], preferred_element_type=F32) + b2[...], 0.0)
    hn = h[...] + jnp.dot(a, w3[...], preferred_element_type=F32) + b3[...]
    hn_out[...] = hn
    ps_out[...] = jnp.dot(hn, wea[...], preferred_element_type=F32)
    pd_out[...] = jnp.dot(hn, web[...], preferred_element_type=F32)


def _node_call(h, am, ac, im, ic, w1a, w1b, w1c, b1, w2, b2, w3, b3, wea, web):
    n, d = h.shape
    row = pl.BlockSpec((BR, d), lambda i: (i, 0))
    prow0 = pl.BlockSpec((1, BR, d), lambda i: (0, i, 0))
    prow1 = pl.BlockSpec((1, BR, d), lambda i: (1, i, 0))
    mat = pl.BlockSpec((d, d), lambda i: (0, 0))
    vec = pl.BlockSpec((1, d), lambda i: (0, 0))
    return pl.pallas_call(
        _node_body,
        grid=(n // BR,),
        in_specs=[row, prow0, prow1, prow0, prow1, row, row,
                  mat, mat, mat, vec, mat, vec, mat, vec, mat, mat],
        out_specs=[row, row, row],
        out_shape=[jax.ShapeDtypeStruct((n, d), F32)] * 3,
    )(h, am, am, ac, ac, im, ic, w1a, w1b, w1c, b1, w2, b2, w3, b3, wea, web)


def _sc_mesh():
    return plsc.VectorSubcoreMesh(core_axis_name="c", subcore_axis_name="s",
                                  num_cores=NC, num_subcores=NS)


def _gather_pipe(np_rows, idx_hbm, tbl, out, hbm_dummy, ix_a, ix_b, rv_a, rv_b,
                 gs_a, gs_b, ws_a, ws_b, tid):
    e_total = out.shape[0]
    ept = e_total // NS
    nch = ept // CHUNK
    base0 = tid * ept
    dummy = out.at[pl.ds(0, CHUNK)]

    def idx_load(j, ix):
        pltpu.sync_copy(idx_hbm.at[pl.ds(base0 + j * CHUNK, CHUNK)], ix)

    def fire_gather(ix, rv, sem):
        pltpu.async_copy(tbl.at[ix], rv, sem)

    def wait1(sem, rv):
        pltpu.make_async_copy(hbm_dummy, rv, sem).wait()

    def fire_write(j, rv, sem):
        pltpu.async_copy(rv, out.at[pl.ds(base0 + j * CHUNK, CHUNK)], sem)

    def wait_w(sem, rv):
        pltpu.make_async_copy(dummy, rv, sem).wait()

    idx_load(0, ix_a)
    fire_gather(ix_a, rv_a, gs_a)
    idx_load(1, ix_b)

    def body(g, _):
        j = 2 * g
        wait1(gs_a, rv_a)

        @pl.when(g > 0)
        def _():
            wait_w(ws_b, rv_b)
        fire_gather(ix_b, rv_b, gs_b)
        fire_write(j, rv_a, ws_a)

        @pl.when(j + 2 < nch)
        def _():
            idx_load(j + 2, ix_a)
        wait1(gs_b, rv_b)
        wait_w(ws_a, rv_a)

        @pl.when(j + 2 < nch)
        def _():
            fire_gather(ix_a, rv_a, gs_a)
        fire_write(j + 1, rv_b, ws_b)

        @pl.when(j + 3 < nch)
        def _():
            idx_load(j + 3, ix_b)
        return 0

    lax.fori_loop(0, nch // 2, body, 0)
    wait_w(ws_b, rv_b)


def _gather_body(np_rows, tbl_rows, ps, pd, src, dst, g1, g2,
                 ix_a, ix_b, rv_a, rv_b, tbl, gs_a, gs_b, ws_a, ws_b):
    core = lax.axis_index("c")
    tid = lax.axis_index("s")
    trows = tbl_rows // NS

    @pl.when(core == 0)
    def _():
        pltpu.sync_copy(ps.at[pl.ds(tid * trows, trows)],
                        tbl.at[pl.ds(tid * trows, trows), :])

    @pl.when(core == 1)
    def _():
        pltpu.sync_copy(pd.at[pl.ds(tid * trows, trows)],
                        tbl.at[pl.ds(tid * trows, trows), :])
    plsc.subcore_barrier()

    hbm_dummy = ps.at[pl.ds(0, CHUNK)]

    @pl.when(core == 0)
    def _():
        _gather_pipe(np_rows, src, tbl, g1, hbm_dummy, ix_a, ix_b, rv_a, rv_b,
                     gs_a, gs_b, ws_a, ws_b, tid)

    @pl.when(core == 1)
    def _():
        _gather_pipe(np_rows, dst, tbl, g2, hbm_dummy, ix_a, ix_b, rv_a, rv_b,
                     gs_a, gs_b, ws_a, ws_b, tid)


def _gather_call(e_total, d, np_rows, tbl_rows):
    return pl.kernel(
        functools.partial(_gather_body, np_rows, tbl_rows),
        out_type=[jax.ShapeDtypeStruct((e_total, d), F32),
                  jax.ShapeDtypeStruct((e_total, d), F32)],
        mesh=_sc_mesh(),
        scratch_types=[
            pltpu.VMEM((CHUNK,), jnp.int32),
            pltpu.VMEM((CHUNK,), jnp.int32),
            pltpu.VMEM((CHUNK, d), F32),
            pltpu.VMEM((CHUNK, d), F32),
            pltpu.VMEM_SHARED((tbl_rows, d), F32),
            pltpu.SemaphoreType.DMA,
            pltpu.SemaphoreType.DMA,
            pltpu.SemaphoreType.DMA,
            pltpu.SemaphoreType.DMA,
        ],
    )


def _zero_block(buf):
    def zr(r, _):
        for c in range(8):
            buf[r, pl.ds(c * 16, 16)] = jnp.zeros((16,), F32)
        return 0
    lax.fori_loop(0, CHUNK, zr, 0)


def _ones_block(buf):
    def zr(r, _):
        for c in range(8):
            buf[r, pl.ds(c * 16, 16)] = jnp.ones((16,), F32)
        return 0
    lax.fori_loop(0, CHUNK, zr, 0)


def _scatter_body(base_off, n_edges, np_rows, acc_rows, e, dsts, out,
                  ix_a, ix_b, rv_a, rv_b, acc, ls_a, ls_b, ss_a, ss_b):
    core = lax.axis_index("c")
    tid = lax.axis_index("s")
    ept = n_edges // (NC * NS)
    nch = ept // CHUNK
    zrows = acc_rows // NS
    base0 = core * (n_edges // NC) + tid * ept
    dummy_i = dsts.at[pl.ds(0, CHUNK)]
    dummy_r = e.at[pl.ds(0, CHUNK), :]

    _zero_block(rv_a)
    nz, rem = zrows // CHUNK, zrows % CHUNK
    for k in range(nz):
        pltpu.sync_copy(rv_a, acc.at[pl.ds(tid * zrows + k * CHUNK, CHUNK), :])
    if rem:
        pltpu.sync_copy(rv_a.at[pl.ds(0, rem), :],
                        acc.at[pl.ds(tid * zrows + nz * CHUNK, rem), :])
    plsc.subcore_barrier()

    def load(j, ix, rv, sem):
        pltpu.async_copy(dsts.at[pl.ds(base0 + j * CHUNK, CHUNK)], ix, sem)
        pltpu.async_copy(e.at[pl.ds(base_off + base0 + j * CHUNK, CHUNK), :], rv, sem)

    def wait_load(sem, ix, rv):
        pltpu.make_async_copy(dummy_i, ix, sem).wait()
        pltpu.make_async_copy(dummy_r, rv, sem).wait()

    def wait_add(sem, rv):
        pltpu.make_async_copy(dummy_r, rv, sem).wait()

    load(0, ix_a, rv_a, ls_a)

    def body(g, _):
        j = 2 * g
        wait_load(ls_a, ix_a, rv_a)

        @pl.when(g > 0)
        def _():
            wait_add(ss_b, rv_b)
        load(j + 1, ix_b, rv_b, ls_b)
        pltpu.async_copy(rv_a, acc.at[ix_a], ss_a, add=True)
        wait_load(ls_b, ix_b, rv_b)
        wait_add(ss_a, rv_a)

        @pl.when(j + 2 < nch)
        def _():
            load(j + 2, ix_a, rv_a, ls_a)
        pltpu.async_copy(rv_b, acc.at[ix_b], ss_b, add=True)
        return 0

    lax.fori_loop(0, nch // 2, body, 0)
    wait_add(ss_b, rv_b)
    plsc.subcore_barrier()

    pltpu.sync_copy(acc.at[pl.ds(tid * zrows, zrows), :],
                    out.at[core, pl.ds(tid * zrows, zrows), :])


def _scatter_call(base_off, n_edges, d, np_rows, acc_rows):
    return pl.kernel(
        functools.partial(_scatter_body, base_off, n_edges, np_rows, acc_rows),
        out_type=jax.ShapeDtypeStruct((NC, np_rows, d), F32),
        mesh=_sc_mesh(),
        scratch_types=[
            pltpu.VMEM((CHUNK,), jnp.int32),
            pltpu.VMEM((CHUNK,), jnp.int32),
            pltpu.VMEM((CHUNK, 128), F32),
            pltpu.VMEM((CHUNK, 128), F32),
            pltpu.VMEM_SHARED((acc_rows, 128), F32),
            pltpu.SemaphoreType.DMA,
            pltpu.SemaphoreType.DMA,
            pltpu.SemaphoreType.DMA,
            pltpu.SemaphoreType.DMA,
        ],
    )


def _degree_body(n_edges, np_rows, acc_rows, dsts, out, idx_v, ones_v, zbuf, acc):
    core = lax.axis_index("c")
    tid = lax.axis_index("s")
    ept = n_edges // (NC * NS)
    nch = ept // CHUNK
    zrows = acc_rows // NS

    _zero_block(zbuf)
    nz, rem = zrows // CHUNK, zrows % CHUNK
    for k in range(nz):
        pltpu.sync_copy(zbuf, acc.at[pl.ds(tid * zrows + k * CHUNK, CHUNK), :])
    if rem:
        pltpu.sync_copy(zbuf.at[pl.ds(0, rem), :],
                        acc.at[pl.ds(tid * zrows + nz * CHUNK, rem), :])
    _ones_block(ones_v)
    plsc.subcore_barrier()

    def body(i, _):
        base = core * (n_edges // NC) + tid * ept + i * CHUNK
        pltpu.sync_copy(dsts.at[pl.ds(base, CHUNK)], idx_v)
        pltpu.sync_copy(ones_v, acc.at[idx_v], add=True)
        return 0

    lax.fori_loop(0, nch, body, 0)
    plsc.subcore_barrier()

    pltpu.sync_copy(acc.at[pl.ds(tid * zrows, zrows), :],
                    out.at[core, pl.ds(tid * zrows, zrows), :])


def _degree_call(n_edges, d, np_rows, acc_rows):
    return pl.kernel(
        functools.partial(_degree_body, n_edges, np_rows, acc_rows),
        out_type=jax.ShapeDtypeStruct((NC, np_rows, d), F32),
        mesh=_sc_mesh(),
        scratch_types=[
            pltpu.VMEM((CHUNK,), jnp.int32),
            pltpu.VMEM((CHUNK, 128), F32),
            pltpu.VMEM((CHUNK, 128), F32),
            pltpu.VMEM_SHARED((acc_rows, 128), F32),
        ],
    )


def _invdeg_body(dm0, dm1, dc0, dc1, om, oc):
    om[...] = 1.0 / jnp.clip(dm0[...] + dm1[...], 1.0, None)
    oc[...] = 1.0 / jnp.clip(dc0[...] + dc1[...], 1.0, None)


def _invdeg_call(dm, dc):
    n, d = dm.shape[1:]
    row = pl.BlockSpec((BR, d), lambda i: (i, 0))
    return pl.pallas_call(
        _invdeg_body,
        grid=(n // BR,),
        in_specs=[row, row, row, row],
        out_specs=[row, row],
        out_shape=[jax.ShapeDtypeStruct((n, d), F32)] * 2,
    )(dm[0], dm[1], dc[0], dc[1])


def kernel(x, mesh_edge_index, mesh_edge_attr, contact_edge_index, contact_edge_attr, params):
    n_nodes, node_dim = x.shape
    n_mesh = mesh_edge_attr.shape[0]
    n_contact = contact_edge_attr.shape[0]
    edge_dim = mesh_edge_attr.shape[1]
    d = params["node_enc"][4].shape[1]
    steps = 5

    np_rows = _ceil_to(n_nodes + 1, BR)
    acc_rows = _ceil_to(n_nodes + 1, 128)
    em = _ceil_to(n_mesh, 2 * NW * CHUNK)
    ec = _ceil_to(n_contact, 2 * NW * CHUNK)
    e_total = em + ec

    xp = jnp.pad(x, ((0, np_rows - n_nodes), (0, 0)))
    msrc = jnp.pad(mesh_edge_index[0], (0, em - n_mesh))
    mdst = jnp.pad(mesh_edge_index[1], (0, em - n_mesh), constant_values=n_nodes)
    csrc = jnp.pad(contact_edge_index[0], (0, ec - n_contact))
    cdst = jnp.pad(contact_edge_index[1], (0, ec - n_contact), constant_values=n_nodes)
    src_all = jnp.concatenate([msrc, csrc])
    dst_all = jnp.concatenate([mdst, cdst])
    ea = jnp.concatenate([
        jnp.pad(mesh_edge_attr, ((0, em - n_mesh), (0, 0))),
        jnp.pad(contact_edge_attr, ((0, ec - n_contact), (0, 0))),
    ])

    def stack2(pa, pb):
        out = []
        for wa, wb in zip(pa, pb):
            if wa.ndim == 1:
                wa, wb = wa[None, :], wb[None, :]
            out.append(jnp.stack([wa, wb]))
        return out

    def stack1(p):
        return [a[None][:, None, :] if a.ndim == 1 else a[None] for a in p]

    enc_e_ws = stack2(params["mesh_enc"], params["contact_enc"])
    enc_n_ws = stack1(params["node_enc"])

    we1, be1, we2, be2, we3, be3 = params["edge_mlp"]
    we1a, we1b, we1c = we1[:d], we1[d:2 * d], we1[2 * d:]
    be1r, be2r, be3r = be1[None, :], be2[None, :], be3[None, :]

    wn1, bn1, wn2, bn2, wn3, bn3 = params["node_mlp"]
    wn1a, wn1b, wn1c = wn1[:d], wn1[d:2 * d], wn1[2 * d:]
    bn1r, bn2r, bn3r = bn1[None, :], bn2[None, :], bn3[None, :]

    wd1, bd1, wd2, bd2, wd3, bd3 = params["decoder"]
    out_dim = wd3.shape[1]
    wd3p = jnp.pad(wd3, ((0, 0), (0, d - out_dim)))
    bd3p = jnp.pad(bd3, (0, d - out_dim))
    dec_ws = stack1((wd1, bd1, wd2, bd2, wd3p, bd3p))

    gather = _gather_call(e_total, d, np_rows, acc_rows)
    scatter_m = _scatter_call(0, em, d, np_rows, acc_rows)
    scatter_c = _scatter_call(em, ec, d, np_rows, acc_rows)

    e = _mlp3_call(ea, enc_e_ws, em)
    h = _mlp3_call(xp, enc_n_ws, np_rows)
    ps, pd = _proj_call(h, we1a, we1b)
    degm = _degree_call(em, d, np_rows, acc_rows)(mdst)
    degc = _degree_call(ec, d, np_rows, acc_rows)(cdst)
    invm, invc = _invdeg_call(degm, degc)

    for _ in range(steps):
        g1, g2 = gather(ps, pd, src_all, dst_all)
        e = _edge_call(g1, g2, e, we1c, be1r, we2, be2r, we3, be3r)
        aggm = scatter_m(e, mdst)
        aggc = scatter_c(e, cdst)
        h, ps, pd = _node_call(h, aggm, aggc, invm, invc,
                               wn1a, wn1b, wn1c, bn1r, wn2, bn2r, wn3, bn3r,
                               we1a, we1b)

    dec = _mlp3_call(h, dec_ws, np_rows)
    return dec[:n_nodes, :out_dim]


# speedup vs baseline: 1.2519x; 1.0973x over previous
import functools

import jax
import jax.numpy as jnp
from jax import lax
from jax.experimental import pallas as pl
from jax.experimental.pallas import tpu as pltpu
from jax.experimental.pallas import tpu_sc as plsc

F32 = jnp.float32
NC, NS = 2, 16
NW = NC * NS
CHUNK = 128
BR = 1024


def _ceil_to(x, m):
    return (x + m - 1) // m * m


def _pack_bf16(x):
    k = x.shape[1] // 2
    xi = jax.lax.bitcast_convert_type(x, jnp.int32)
    lo = jax.lax.shift_right_logical(xi[:, :k] + 0x8000, 16)
    hi = (xi[:, k:] + 0x8000) & jnp.int32(-65536)
    return lo | hi


def _unpack_bf16(gi):
    a = jax.lax.bitcast_convert_type(gi << 16, F32)
    b = jax.lax.bitcast_convert_type(gi & jnp.int32(-65536), F32)
    return jnp.concatenate([a, b], axis=1)


def _mlp3_body(x, w1, b1, w2, b2, w3, b3, out):
    h = jnp.maximum(jnp.dot(x[...], w1[0], preferred_element_type=F32) + b1[0], 0.0)
    h = jnp.maximum(jnp.dot(h, w2[0], preferred_element_type=F32) + b2[0], 0.0)
    out[...] = jnp.dot(h, w3[0], preferred_element_type=F32) + b3[0]


def _mlp3_call(x, ws, n_first):
    n, din = x.shape
    dout = ws[4].shape[-1]
    grid = (n // BR,)
    nb_first = n_first // BR

    def wsel(i):
        return (jnp.where(i < nb_first, 0, 1), 0, 0)

    wspecs = []
    for a in ws:
        s, d0, d1 = a.shape
        wspecs.append(pl.BlockSpec((1, d0, d1), wsel if s > 1 else (lambda i: (0, 0, 0))))
    return pl.pallas_call(
        _mlp3_body,
        grid=grid,
        in_specs=[pl.BlockSpec((BR, din), lambda i: (i, 0))] + wspecs,
        out_specs=pl.BlockSpec((BR, dout), lambda i: (i, 0)),
        out_shape=jax.ShapeDtypeStruct((n, dout), F32),
    )(x, *ws)


def _proj_body(h, wa, wb, ps, pd):
    ps[...] = jnp.dot(h[...], wa[...], preferred_element_type=F32)
    pd[...] = jnp.dot(h[...], wb[...], preferred_element_type=F32)


def _proj_call(h, wa, wb):
    n, d = h.shape
    return pl.pallas_call(
        _proj_body,
        grid=(n // BR,),
        in_specs=[pl.BlockSpec((BR, d), lambda i: (i, 0)),
                  pl.BlockSpec((d, d), lambda i: (0, 0)),
                  pl.BlockSpec((d, d), lambda i: (0, 0))],
        out_specs=[pl.BlockSpec((BR, d), lambda i: (i, 0)),
                   pl.BlockSpec((BR, d), lambda i: (i, 0))],
        out_shape=[jax.ShapeDtypeStruct((n, d), F32),
                   jax.ShapeDtypeStruct((n, d), F32)],
    )(h, wa, wb)


def _edge_body(g1, g2, e, w1c, b1, w2, b2, w3, b3, out):
    a = jnp.maximum(g1[...] + g2[...] + b1[...]
                    + jnp.dot(e[...], w1c[...], preferred_element_type=F32), 0.0)
    a = jnp.maximum(jnp.dot(a, w2[...], preferred_element_type=F32) + b2[...], 0.0)
    out[...] = e[...] + jnp.dot(a, w3[...], preferred_element_type=F32) + b3[...]


def _edge_call(g1, g2, e, w1c, b1, w2, b2, w3, b3):
    n, d = e.shape
    row = pl.BlockSpec((BR, d), lambda i: (i, 0))
    mat = pl.BlockSpec((d, d), lambda i: (0, 0))
    vec = pl.BlockSpec((1, d), lambda i: (0, 0))
    return pl.pallas_call(
        _edge_body,
        grid=(n // BR,),
        in_specs=[row, row, row, mat, vec, mat, vec, mat, vec],
        out_specs=row,
        out_shape=jax.ShapeDtypeStruct((n, d), F32),
    )(g1, g2, e, w1c, b1, w2, b2, w3, b3)


def _node_body(h, am0, am1, ac0, ac1, im, ic, w1a, w1b, w1c, b1, w2, b2, w3, b3,
               wea, web, hn_out, ps_out, pd_out):
    am_n = (am0[0] + am1[0]) * im[...]
    ac_n = (ac0[0] + ac1[0]) * ic[...]
    a = (jnp.dot(h[...], w1a[...], preferred_element_type=F32)
         + jnp.dot(am_n, w1b[...], preferred_element_type=F32)
         + jnp.dot(ac_n, w1c[...], preferred_element_type=F32) + b1[...])
    a = jnp.maximum(a, 0.0)
    a = jnp.maximum(jnp.dot(a, w2[...], preferred_element_type=F32) + b2[...], 0.0)
    hn = h[...] + jnp.dot(a, w3[...], preferred_element_type=F32) + b3[...]
    hn_out[...] = hn
    ps_out[...] = jnp.dot(hn, wea[...], preferred_element_type=F32)
    pd_out[...] = jnp.dot(hn, web[...], preferred_element_type=F32)


def _node_call(h, am, ac, im, ic, w1a, w1b, w1c, b1, w2, b2, w3, b3, wea, web):
    n, d = h.shape
    row = pl.BlockSpec((BR, d), lambda i: (i, 0))
    prow0 = pl.BlockSpec((1, BR, d), lambda i: (0, i, 0))
    prow1 = pl.BlockSpec((1, BR, d), lambda i: (1, i, 0))
    mat = pl.BlockSpec((d, d), lambda i: (0, 0))
    vec = pl.BlockSpec((1, d), lambda i: (0, 0))
    return pl.pallas_call(
        _node_body,
        grid=(n // BR,),
        in_specs=[row, prow0, prow1, prow0, prow1, row, row,
                  mat, mat, mat, vec, mat, vec, mat, vec, mat, mat],
        out_specs=[row, row, row],
        out_shape=[jax.ShapeDtypeStruct((n, d), F32)] * 3,
    )(h, am, am, ac, ac, im, ic, w1a, w1b, w1c, b1, w2, b2, w3, b3, wea, web)


def _sc_mesh():
    return plsc.VectorSubcoreMesh(core_axis_name="c", subcore_axis_name="s",
                                  num_cores=NC, num_subcores=NS)


def _gather_pipe(np_rows, idx_hbm, tbl, out, hbm_dummy, ix_a, ix_b, rv_a, rv_b,
                 gs_a, gs_b, ws_a, ws_b, tid):
    e_total = out.shape[0]
    ept = e_total // NS
    nch = ept // CHUNK
    base0 = tid * ept
    dummy = out.at[pl.ds(0, CHUNK)]

    def idx_load(j, ix):
        pltpu.sync_copy(idx_hbm.at[pl.ds(base0 + j * CHUNK, CHUNK)], ix)

    def fire_gather(ix, rv, sem):
        pltpu.async_copy(tbl.at[ix], rv, sem)

    def wait1(sem, rv):
        pltpu.make_async_copy(hbm_dummy, rv, sem).wait()

    def fire_write(j, rv, sem):
        pltpu.async_copy(rv, out.at[pl.ds(base0 + j * CHUNK, CHUNK)], sem)

    def wait_w(sem, rv):
        pltpu.make_async_copy(dummy, rv, sem).wait()

    idx_load(0, ix_a)
    fire_gather(ix_a, rv_a, gs_a)
    idx_load(1, ix_b)

    def body(g, _):
        j = 2 * g
        wait1(gs_a, rv_a)

        @pl.when(g > 0)
        def _():
            wait_w(ws_b, rv_b)
        fire_gather(ix_b, rv_b, gs_b)
        fire_write(j, rv_a, ws_a)

        @pl.when(j + 2 < nch)
        def _():
            idx_load(j + 2, ix_a)
        wait1(gs_b, rv_b)
        wait_w(ws_a, rv_a)

        @pl.when(j + 2 < nch)
        def _():
            fire_gather(ix_a, rv_a, gs_a)
        fire_write(j + 1, rv_b, ws_b)

        @pl.when(j + 3 < nch)
        def _():
            idx_load(j + 3, ix_b)
        return 0

    lax.fori_loop(0, nch // 2, body, 0)
    wait_w(ws_b, rv_b)


def _gather_body(np_rows, tbl_rows, ps, pd, src, dst, g1, g2,
                 ix_a, ix_b, rv_a, rv_b, tbl, gs_a, gs_b, ws_a, ws_b):
    core = lax.axis_index("c")
    tid = lax.axis_index("s")
    trows = tbl_rows // NS

    @pl.when(core == 0)
    def _():
        pltpu.sync_copy(ps.at[pl.ds(tid * trows, trows)],
                        tbl.at[pl.ds(tid * trows, trows), :])

    @pl.when(core == 1)
    def _():
        pltpu.sync_copy(pd.at[pl.ds(tid * trows, trows)],
                        tbl.at[pl.ds(tid * trows, trows), :])
    plsc.subcore_barrier()

    hbm_dummy = ps.at[pl.ds(0, CHUNK)]

    @pl.when(core == 0)
    def _():
        _gather_pipe(np_rows, src, tbl, g1, hbm_dummy, ix_a, ix_b, rv_a, rv_b,
                     gs_a, gs_b, ws_a, ws_b, tid)

    @pl.when(core == 1)
    def _():
        _gather_pipe(np_rows, dst, tbl, g2, hbm_dummy, ix_a, ix_b, rv_a, rv_b,
                     gs_a, gs_b, ws_a, ws_b, tid)


def _gather_call(e_total, d, np_rows, tbl_rows):
    return pl.kernel(
        functools.partial(_gather_body, np_rows, tbl_rows),
        out_type=[jax.ShapeDtypeStruct((e_total, d), F32),
                  jax.ShapeDtypeStruct((e_total, d), F32)],
        mesh=_sc_mesh(),
        scratch_types=[
            pltpu.VMEM((CHUNK,), jnp.int32),
            pltpu.VMEM((CHUNK,), jnp.int32),
            pltpu.VMEM((CHUNK, d), F32),
            pltpu.VMEM((CHUNK, d), F32),
            pltpu.VMEM_SHARED((tbl_rows, d), F32),
            pltpu.SemaphoreType.DMA,
            pltpu.SemaphoreType.DMA,
            pltpu.SemaphoreType.DMA,
            pltpu.SemaphoreType.DMA,
        ],
    )


def _zero_block(buf):
    def zr(r, _):
        for c in range(8):
            buf[r, pl.ds(c * 16, 16)] = jnp.zeros((16,), F32)
        return 0
    lax.fori_loop(0, CHUNK, zr, 0)


def _ones_block(buf):
    def zr(r, _):
        for c in range(8):
            buf[r, pl.ds(c * 16, 16)] = jnp.ones((16,), F32)
        return 0
    lax.fori_loop(0, CHUNK, zr, 0)


def _scatter_body(base_off, n_edges, np_rows, acc_rows, e, dsts, out,
                  ix_a, ix_b, rv_a, rv_b, acc, ls_a, ls_b, ss_a, ss_b):
    core = lax.axis_index("c")
    tid = lax.axis_index("s")
    ept = n_edges // (NC * NS)
    nch = ept // CHUNK
    zrows = acc_rows // NS
    base0 = core * (n_edges // NC) + tid * ept
    dummy_i = dsts.at[pl.ds(0, CHUNK)]
    dummy_r = e.at[pl.ds(0, CHUNK), :]

    _zero_block(rv_a)
    nz, rem = zrows // CHUNK, zrows % CHUNK
    for k in range(nz):
        pltpu.sync_copy(rv_a, acc.at[pl.ds(tid * zrows + k * CHUNK, CHUNK), :])
    if rem:
        pltpu.sync_copy(rv_a.at[pl.ds(0, rem), :],
                        acc.at[pl.ds(tid * zrows + nz * CHUNK, rem), :])
    plsc.subcore_barrier()

    def load(j, ix, rv, sem):
        pltpu.async_copy(dsts.at[pl.ds(base0 + j * CHUNK, CHUNK)], ix, sem)
        pltpu.async_copy(e.at[pl.ds(base_off + base0 + j * CHUNK, CHUNK), :], rv, sem)

    def wait_load(sem, ix, rv):
        pltpu.make_async_copy(dummy_i, ix, sem).wait()
        pltpu.make_async_copy(dummy_r, rv, sem).wait()

    def wait_add(sem, rv):
        pltpu.make_async_copy(dummy_r, rv, sem).wait()

    load(0, ix_a, rv_a, ls_a)

    def body(g, _):
        j = 2 * g
        wait_load(ls_a, ix_a, rv_a)

        @pl.when(g > 0)
        def _():
            wait_add(ss_b, rv_b)
        load(j + 1, ix_b, rv_b, ls_b)
        pltpu.async_copy(rv_a, acc.at[ix_a], ss_a, add=True)
        wait_load(ls_b, ix_b, rv_b)
        wait_add(ss_a, rv_a)

        @pl.when(j + 2 < nch)
        def _():
            load(j + 2, ix_a, rv_a, ls_a)
        pltpu.async_copy(rv_b, acc.at[ix_b], ss_b, add=True)
        return 0

    lax.fori_loop(0, nch // 2, body, 0)
    wait_add(ss_b, rv_b)
    plsc.subcore_barrier()

    pltpu.sync_copy(acc.at[pl.ds(tid * zrows, zrows), :],
                    out.at[core, pl.ds(tid * zrows, zrows), :])


def _scatter_call(base_off, n_edges, d, np_rows, acc_rows):
    return pl.kernel(
        functools.partial(_scatter_body, base_off, n_edges, np_rows, acc_rows),
        out_type=jax.ShapeDtypeStruct((NC, np_rows, d), F32),
        mesh=_sc_mesh(),
        scratch_types=[
            pltpu.VMEM((CHUNK,), jnp.int32),
            pltpu.VMEM((CHUNK,), jnp.int32),
            pltpu.VMEM((CHUNK, 128), F32),
            pltpu.VMEM((CHUNK, 128), F32),
            pltpu.VMEM_SHARED((acc_rows, 128), F32),
            pltpu.SemaphoreType.DMA,
            pltpu.SemaphoreType.DMA,
            pltpu.SemaphoreType.DMA,
            pltpu.SemaphoreType.DMA,
        ],
    )


def _degree_phase(n_edges, acc_rows, dsts, out, idx_v, ones_v, zbuf, acc, core, tid):
    ept = n_edges // (NC * NS)
    nch = ept // CHUNK
    zrows = acc_rows // NS

    nz, rem = zrows // CHUNK, zrows % CHUNK
    for k in range(nz):
        pltpu.sync_copy(zbuf, acc.at[pl.ds(tid * zrows + k * CHUNK, CHUNK), :])
    if rem:
        pltpu.sync_copy(zbuf.at[pl.ds(0, rem), :],
                        acc.at[pl.ds(tid * zrows + nz * CHUNK, rem), :])
    plsc.subcore_barrier()

    def body(i, _):
        base = core * (n_edges // NC) + tid * ept + i * CHUNK
        pltpu.sync_copy(dsts.at[pl.ds(base, CHUNK)], idx_v)
        pltpu.sync_copy(ones_v, acc.at[idx_v], add=True)
        return 0

    lax.fori_loop(0, nch, body, 0)
    plsc.subcore_barrier()

    pltpu.sync_copy(acc.at[pl.ds(tid * zrows, zrows), :],
                    out.at[core, pl.ds(tid * zrows, zrows), :])


def _degree_body(em, ec, acc_rows, mdst, cdst, outm, outc, idx_v, ones_v, zbuf, acc):
    core = lax.axis_index("c")
    tid = lax.axis_index("s")
    _zero_block(zbuf)
    _ones_block(ones_v)
    _degree_phase(em, acc_rows, mdst, outm, idx_v, ones_v, zbuf, acc, core, tid)
    _degree_phase(ec, acc_rows, cdst, outc, idx_v, ones_v, zbuf, acc, core, tid)


def _degree_call(em, ec, d, np_rows, acc_rows):
    return pl.kernel(
        functools.partial(_degree_body, em, ec, acc_rows),
        out_type=[jax.ShapeDtypeStruct((NC, np_rows, d), F32),
                  jax.ShapeDtypeStruct((NC, np_rows, d), F32)],
        mesh=_sc_mesh(),
        scratch_types=[
            pltpu.VMEM((CHUNK,), jnp.int32),
            pltpu.VMEM((CHUNK, 128), F32),
            pltpu.VMEM((CHUNK, 128), F32),
            pltpu.VMEM_SHARED((acc_rows, 128), F32),
        ],
    )


def _invdeg_body(dm0, dm1, dc0, dc1, om, oc):
    om[...] = 1.0 / jnp.clip(dm0[...] + dm1[...], 1.0, None)
    oc[...] = 1.0 / jnp.clip(dc0[...] + dc1[...], 1.0, None)


def _invdeg_call(dm, dc):
    n, d = dm.shape[1:]
    row = pl.BlockSpec((BR, d), lambda i: (i, 0))
    return pl.pallas_call(
        _invdeg_body,
        grid=(n // BR,),
        in_specs=[row, row, row, row],
        out_specs=[row, row],
        out_shape=[jax.ShapeDtypeStruct((n, d), F32)] * 2,
    )(dm[0], dm[1], dc[0], dc[1])


def kernel(x, mesh_edge_index, mesh_edge_attr, contact_edge_index, contact_edge_attr, params):
    n_nodes, node_dim = x.shape
    n_mesh = mesh_edge_attr.shape[0]
    n_contact = contact_edge_attr.shape[0]
    edge_dim = mesh_edge_attr.shape[1]
    d = params["node_enc"][4].shape[1]
    steps = 5

    np_rows = _ceil_to(n_nodes + 1, BR)
    acc_rows = _ceil_to(n_nodes + 1, 128)
    em = _ceil_to(n_mesh, 2 * NW * CHUNK)
    ec = _ceil_to(n_contact, 2 * NW * CHUNK)
    e_total = em + ec

    xp = jnp.pad(x, ((0, np_rows - n_nodes), (0, 0)))
    msrc = jnp.pad(mesh_edge_index[0], (0, em - n_mesh))
    mdst = jnp.pad(mesh_edge_index[1], (0, em - n_mesh), constant_values=n_nodes)
    csrc = jnp.pad(contact_edge_index[0], (0, ec - n_contact))
    cdst = jnp.pad(contact_edge_index[1], (0, ec - n_contact), constant_values=n_nodes)
    eam = jnp.pad(mesh_edge_attr, ((0, em - n_mesh), (0, 0)))
    eac = jnp.pad(contact_edge_attr, ((0, ec - n_contact), (0, 0)))

    def stack2(pa, pb):
        out = []
        for wa, wb in zip(pa, pb):
            if wa.ndim == 1:
                wa, wb = wa[None, :], wb[None, :]
            out.append(jnp.stack([wa, wb]))
        return out

    def stack1(p):
        return [a[None][:, None, :] if a.ndim == 1 else a[None] for a in p]

    enc_m_ws = stack1(params["mesh_enc"])
    enc_c_ws = stack1(params["contact_enc"])
    enc_n_ws = stack1(params["node_enc"])

    we1, be1, we2, be2, we3, be3 = params["edge_mlp"]
    we1a, we1b, we1c = we1[:d], we1[d:2 * d], we1[2 * d:]
    be1r, be2r, be3r = be1[None, :], be2[None, :], be3[None, :]

    wn1, bn1, wn2, bn2, wn3, bn3 = params["node_mlp"]
    wn1a, wn1b, wn1c = wn1[:d], wn1[d:2 * d], wn1[2 * d:]
    bn1r, bn2r, bn3r = bn1[None, :], bn2[None, :], bn3[None, :]

    wd1, bd1, wd2, bd2, wd3, bd3 = params["decoder"]
    out_dim = wd3.shape[1]
    wd3p = jnp.pad(wd3, ((0, 0), (0, d - out_dim)))
    bd3p = jnp.pad(bd3, (0, d - out_dim))
    dec_ws = stack1((wd1, bd1, wd2, bd2, wd3p, bd3p))

    gather_m = _gather_call(em, d, np_rows, acc_rows)
    gather_c = _gather_call(ec, d, np_rows, acc_rows)
    scatter_m = _scatter_call(0, em, d, np_rows, acc_rows)
    scatter_c = _scatter_call(0, ec, d, np_rows, acc_rows)

    e_m = _mlp3_call(eam, enc_m_ws, em)
    e_c = _mlp3_call(eac, enc_c_ws, ec)
    h = _mlp3_call(xp, enc_n_ws, np_rows)
    ps, pd = _proj_call(h, we1a, we1b)
    degm, degc = _degree_call(em, ec, d, np_rows, acc_rows)(mdst, cdst)
    invm, invc = _invdeg_call(degm, degc)

    for _ in range(steps):
        g1m, g2m = gather_m(ps, pd, msrc, mdst)
        g1c, g2c = gather_c(ps, pd, csrc, cdst)
        e_m = _edge_call(g1m, g2m, e_m, we1c, be1r, we2, be2r, we3, be3r)
        e_c = _edge_call(g1c, g2c, e_c, we1c, be1r, we2, be2r, we3, be3r)
        aggm = scatter_m(e_m, mdst)
        aggc = scatter_c(e_c, cdst)
        h, ps, pd = _node_call(h, aggm, aggc, invm, invc,
                               wn1a, wn1b, wn1c, bn1r, wn2, bn2r, wn3, bn3r,
                               we1a, we1b)

    dec = _mlp3_call(h, dec_ws, np_rows)
    return dec[:n_nodes, :out_dim]
